# Initial kernel scaffold; baseline (speedup 1.0000x reference)
import jax
import jax.numpy as jnp
from jax.experimental import pallas as pl


def kernel(x, adj_t, W, b):
    raise NotImplementedError("write your pallas kernel here")


# speedup vs baseline: 5.4284x; 5.4284x over previous
import functools

import jax
import jax.numpy as jnp
from jax import lax
from jax.experimental import pallas as pl
from jax.experimental.pallas import tpu as pltpu
from jax.experimental.pallas import tpu_sc as plsc

N = 10000
D = 128
E = 320000
NC = 2
NS = 16
NW = NC * NS
E_PER_TILE = E // NW
K = 80
CHUNKS = E_PER_TILE // K
R_MAIN = 624
ZR = 16


def _sc_body(x_hbm, src_hbm, dst_hbm, part_hbm,
             src_v, dst_v, rows_v, zero_v, acc_sh, sem):
    c = lax.axis_index("c")
    s = lax.axis_index("s")

    for r in range(ZR):
        for j in range(D // 16):
            zero_v[r, pl.ds(j * 16, 16)] = jnp.zeros((16,), jnp.float32)

    def zcopy(i, carry):
        pltpu.sync_copy(zero_v, acc_sh.at[pl.ds(s * R_MAIN + i * ZR, ZR)])
        return carry
    lax.fori_loop(0, R_MAIN // ZR, zcopy, 0)

    @pl.when(s == NS - 1)
    def _zero_tail():
        pltpu.sync_copy(zero_v, acc_sh.at[pl.ds(N - ZR, ZR)])

    plsc.subcore_barrier()

    wid = c * NS + s
    base_e = wid * E_PER_TILE

    def body(i, carry):
        off = base_e + i * K
        pltpu.sync_copy(src_hbm.at[pl.ds(off, K)], src_v)
        pltpu.async_copy(x_hbm.at[src_v], rows_v, sem).wait()
        pltpu.sync_copy(dst_hbm.at[pl.ds(off, K)], dst_v)
        pltpu.sync_copy(rows_v, acc_sh.at[dst_v], add=True)
        return carry
    lax.fori_loop(0, CHUNKS, body, 0)

    plsc.subcore_barrier()
    pltpu.sync_copy(
        acc_sh.at[pl.ds(s * R_MAIN, R_MAIN)],
        part_hbm.at[pl.ds(c * N + s * R_MAIN, R_MAIN)])

    @pl.when(s == NS - 1)
    def _write_tail():
        pltpu.sync_copy(acc_sh.at[pl.ds(N - ZR, ZR)],
                        part_hbm.at[pl.ds(c * N + N - ZR, ZR)])


_sc_aggregate = functools.partial(
    pl.kernel,
    mesh=plsc.VectorSubcoreMesh(core_axis_name="c", subcore_axis_name="s"),
    out_type=jax.ShapeDtypeStruct((NC * N, D), jnp.float32),
    scratch_types=[
        pltpu.VMEM((K,), jnp.int32),
        pltpu.VMEM((K,), jnp.int32),
        pltpu.VMEM((K, D), jnp.float32),
        pltpu.VMEM((ZR, D), jnp.float32),
        pltpu.VMEM_SHARED((N, D), jnp.float32),
        pltpu.SemaphoreType.DMA,
    ],
)(_sc_body)


def _mm_body(p0_ref, p1_ref, w_ref, b_ref, o_ref):
    acc = p0_ref[...] + p1_ref[...]
    o_ref[...] = (
        jnp.dot(acc, w_ref[...], preferred_element_type=jnp.float32)
        + b_ref[...]
    )


_BM = 1000


def _mm(p0, p1, W, b2d):
    return pl.pallas_call(
        _mm_body,
        grid=(N // _BM,),
        in_specs=[
            pl.BlockSpec((_BM, D), lambda i: (i, 0)),
            pl.BlockSpec((_BM, D), lambda i: (i, 0)),
            pl.BlockSpec((D, D), lambda i: (0, 0)),
            pl.BlockSpec((1, D), lambda i: (0, 0)),
        ],
        out_specs=pl.BlockSpec((_BM, D), lambda i: (i, 0)),
        out_shape=jax.ShapeDtypeStruct((N, D), jnp.float32),
    )(p0, p1, W, b2d)


def kernel(x, adj_t, W, b):
    src = adj_t[0]
    dst = adj_t[1]
    partials = _sc_aggregate(x, src, dst)
    p0 = partials[:N]
    p1 = partials[N:]
    return _mm(p0, p1, W, b.reshape(1, D))


# speedup vs baseline: 10.2580x; 1.8897x over previous
import functools

import jax
import jax.numpy as jnp
from jax import lax
from jax.experimental import pallas as pl
from jax.experimental.pallas import tpu as pltpu
from jax.experimental.pallas import tpu_sc as plsc

N = 10000
D = 128
E = 320000
NC = 2
NS = 16
NW = NC * NS
E_PER_TILE = E // NW
K = 80
CHUNKS = E_PER_TILE // K
R_MAIN = 624
ZR = 16
NBUF = 3
MAIN_ITERS = (CHUNKS - 2) // NBUF - 1


def _sc_body(x_hbm, src_hbm, dst_hbm, part_hbm,
             sidx_v, didx_v, rows_v, zero_v, acc_sh,
             sem_z, si0, si1, si2, sg0, sg1, sg2, ss0, ss1, ss2):
    sem_i = (si0, si1, si2)
    sem_g = (sg0, sg1, sg2)
    sem_s = (ss0, ss1, ss2)
    c = lax.axis_index("c")
    s = lax.axis_index("s")

    wid = c * NS + s

    for r in range(ZR):
        for j in range(D // 16):
            zero_v[r, pl.ds(j * 16, 16)] = jnp.zeros((16,), jnp.float32)

    zcopies = []
    for i in range(R_MAIN // ZR):
        zcopies.append(pltpu.async_copy(
            zero_v, acc_sh.at[pl.ds(s * R_MAIN + i * ZR, ZR)], sem_z))

    @pl.when(s == NS - 1)
    def _zero_tail():
        pltpu.async_copy(zero_v, acc_sh.at[pl.ds(N - ZR, ZR)], sem_z).wait()

    for cp in zcopies:
        cp.wait()
    plsc.subcore_barrier()

    base_e = wid * E_PER_TILE

    def idx_fire(chunk, b):
        off = base_e + chunk * K
        pltpu.async_copy(src_hbm.at[pl.ds(off, K)], sidx_v.at[b], sem_i[b])
        pltpu.async_copy(dst_hbm.at[pl.ds(off, K)], didx_v.at[b], sem_i[b])

    def idx_wait(b):
        pltpu.make_async_copy(src_hbm.at[pl.ds(0, K)], sidx_v.at[b],
                              sem_i[b]).wait()
        pltpu.make_async_copy(dst_hbm.at[pl.ds(0, K)], didx_v.at[b],
                              sem_i[b]).wait()

    def gather(b):
        pltpu.async_copy(x_hbm.at[sidx_v.at[b]], rows_v.at[b], sem_g[b])

    def gather_wait(b):
        pltpu.make_async_copy(x_hbm.at[sidx_v.at[b]], rows_v.at[b],
                              sem_g[b]).wait()

    def scatter(b):
        pltpu.async_copy(rows_v.at[b], acc_sh.at[didx_v.at[b]],
                         sem_s[b], add=True)

    def scatter_wait(b):
        pltpu.make_async_copy(rows_v.at[b], acc_sh.at[didx_v.at[b]],
                              sem_s[b]).wait()

    for b in range(NBUF):
        idx_fire(b, b)
    for b in range(NBUF):
        idx_wait(b)
        gather(b)

    def ring(j, carry):
        base = j * NBUF
        for b in range(NBUF):
            gather_wait(b)
            scatter(b)
        for b in range(NBUF):
            scatter_wait(b)
            idx_fire(base + NBUF + b, b)
        for b in range(NBUF):
            idx_wait(b)
            gather(b)
        return carry
    lax.fori_loop(0, MAIN_ITERS, ring, 0)

    for b in range(NBUF):
        gather_wait(b)
        scatter(b)
    for b in range(2):
        scatter_wait(b)
        idx_fire(CHUNKS - 2 + b, b)
    for b in range(2):
        idx_wait(b)
        gather(b)
    for b in range(2):
        gather_wait(b)
        scatter(b)
    for b in range(2):
        scatter_wait(b)
    scatter_wait(2)

    plsc.subcore_barrier()
    pltpu.sync_copy(
        acc_sh.at[pl.ds(s * R_MAIN, R_MAIN)],
        part_hbm.at[pl.ds(c * N + s * R_MAIN, R_MAIN)])

    @pl.when(s == NS - 1)
    def _write_tail():
        pltpu.sync_copy(acc_sh.at[pl.ds(N - ZR, ZR)],
                        part_hbm.at[pl.ds(c * N + N - ZR, ZR)])


_sc_aggregate = functools.partial(
    pl.kernel,
    mesh=plsc.VectorSubcoreMesh(core_axis_name="c", subcore_axis_name="s"),
    out_type=jax.ShapeDtypeStruct((NC * N, D), jnp.float32),
    scratch_types=[
        pltpu.VMEM((NBUF, K), jnp.int32),
        pltpu.VMEM((NBUF, K), jnp.int32),
        pltpu.VMEM((NBUF, K, D), jnp.float32),
        pltpu.VMEM((ZR, D), jnp.float32),
        pltpu.VMEM_SHARED((N, D), jnp.float32),
    ] + [pltpu.SemaphoreType.DMA] * 10,
)(_sc_body)


def _mm_body(p0_ref, p1_ref, w_ref, b_ref, o_ref):
    acc = p0_ref[...] + p1_ref[...]
    o_ref[...] = (
        jnp.dot(acc, w_ref[...], preferred_element_type=jnp.float32)
        + b_ref[...]
    )


_BM = 1000


def _mm(p0, p1, W, b2d):
    return pl.pallas_call(
        _mm_body,
        grid=(N // _BM,),
        in_specs=[
            pl.BlockSpec((_BM, D), lambda i: (i, 0)),
            pl.BlockSpec((_BM, D), lambda i: (i, 0)),
            pl.BlockSpec((D, D), lambda i: (0, 0)),
            pl.BlockSpec((1, D), lambda i: (0, 0)),
        ],
        out_specs=pl.BlockSpec((_BM, D), lambda i: (i, 0)),
        out_shape=jax.ShapeDtypeStruct((N, D), jnp.float32),
    )(p0, p1, W, b2d)


def kernel(x, adj_t, W, b):
    partials = _sc_aggregate(x, adj_t[0], adj_t[1])
    p0 = partials[:N]
    p1 = partials[N:]
    return _mm(p0, p1, W, b.reshape(1, D))


# speedup vs baseline: 11.4513x; 1.1163x over previous
import functools

import jax
import jax.numpy as jnp
from jax import lax
from jax.experimental import pallas as pl
from jax.experimental.pallas import tpu as pltpu
from jax.experimental.pallas import tpu_sc as plsc

N = 10000
D = 128
E = 320000
NC = 2
NS = 16
NW = NC * NS
E_PER_TILE = E // NW
K = 80
CHUNKS = E_PER_TILE // K
R_MAIN = 624
ZR = 16
NBUF = 3
MAIN_ITERS = (CHUNKS - 2) // NBUF - 1


def _sc_body(x_hbm, adj_hbm, part_hbm,
             sidx_v, didx_v, rows_v, zero_v, acc_sh,
             sem_z, si0, si1, si2, sg0, sg1, sg2, ss0, ss1, ss2):
    sem_i = (si0, si1, si2)
    sem_g = (sg0, sg1, sg2)
    sem_s = (ss0, ss1, ss2)
    c = lax.axis_index("c")
    s = lax.axis_index("s")

    wid = c * NS + s

    for r in range(ZR):
        for j in range(D // 16):
            zero_v[r, pl.ds(j * 16, 16)] = jnp.zeros((16,), jnp.float32)

    zcopies = []
    for i in range(R_MAIN // ZR):
        zcopies.append(pltpu.async_copy(
            zero_v, acc_sh.at[pl.ds(s * R_MAIN + i * ZR, ZR)], sem_z))

    @pl.when(s == NS - 1)
    def _zero_tail():
        pltpu.async_copy(zero_v, acc_sh.at[pl.ds(N - ZR, ZR)], sem_z).wait()

    for cp in zcopies:
        cp.wait()
    plsc.subcore_barrier()

    base_e = wid * E_PER_TILE

    def idx_fire(chunk, b):
        off = base_e + chunk * K
        pltpu.async_copy(adj_hbm.at[pl.ds(off, K)], sidx_v.at[b], sem_i[b])
        pltpu.async_copy(adj_hbm.at[pl.ds(E + off, K)], didx_v.at[b],
                         sem_i[b])

    def idx_wait(b):
        pltpu.make_async_copy(adj_hbm.at[pl.ds(0, K)], sidx_v.at[b],
                              sem_i[b]).wait()
        pltpu.make_async_copy(adj_hbm.at[pl.ds(0, K)], didx_v.at[b],
                              sem_i[b]).wait()

    def gather(b):
        pltpu.async_copy(x_hbm.at[sidx_v.at[b]], rows_v.at[b], sem_g[b])

    def gather_wait(b):
        pltpu.make_async_copy(x_hbm.at[sidx_v.at[b]], rows_v.at[b],
                              sem_g[b]).wait()

    def scatter(b):
        pltpu.async_copy(rows_v.at[b], acc_sh.at[didx_v.at[b]],
                         sem_s[b], add=True)

    def scatter_wait(b):
        pltpu.make_async_copy(rows_v.at[b], acc_sh.at[didx_v.at[b]],
                              sem_s[b]).wait()

    for b in range(NBUF):
        idx_fire(b, b)
    for b in range(NBUF):
        idx_wait(b)
        gather(b)

    def ring(j, carry):
        base = j * NBUF
        for b in range(NBUF):
            gather_wait(b)
            scatter(b)
        for b in range(NBUF):
            scatter_wait(b)
            idx_fire(base + NBUF + b, b)
        for b in range(NBUF):
            idx_wait(b)
            gather(b)
        return carry
    lax.fori_loop(0, MAIN_ITERS, ring, 0)

    for b in range(NBUF):
        gather_wait(b)
        scatter(b)
    for b in range(2):
        scatter_wait(b)
        idx_fire(CHUNKS - 2 + b, b)
    for b in range(2):
        idx_wait(b)
        gather(b)
    for b in range(2):
        gather_wait(b)
        scatter(b)
    for b in range(2):
        scatter_wait(b)
    scatter_wait(2)

    plsc.subcore_barrier()
    pltpu.sync_copy(
        acc_sh.at[pl.ds(s * R_MAIN, R_MAIN)],
        part_hbm.at[pl.ds(c * N + s * R_MAIN, R_MAIN)])

    @pl.when(s == NS - 1)
    def _write_tail():
        pltpu.sync_copy(acc_sh.at[pl.ds(N - ZR, ZR)],
                        part_hbm.at[pl.ds(c * N + N - ZR, ZR)])


_sc_aggregate = functools.partial(
    pl.kernel,
    mesh=plsc.VectorSubcoreMesh(core_axis_name="c", subcore_axis_name="s"),
    out_type=jax.ShapeDtypeStruct((NC * N, D), jnp.float32),
    scratch_types=[
        pltpu.VMEM((NBUF, K), jnp.int32),
        pltpu.VMEM((NBUF, K), jnp.int32),
        pltpu.VMEM((NBUF, K, D), jnp.float32),
        pltpu.VMEM((ZR, D), jnp.float32),
        pltpu.VMEM_SHARED((N, D), jnp.float32),
    ] + [pltpu.SemaphoreType.DMA] * 10,
)(_sc_body)


def _mm_body(p0_ref, p1_ref, w_ref, b_ref, o_ref):
    acc = p0_ref[...] + p1_ref[...]
    o_ref[...] = (
        jnp.dot(acc, w_ref[...], preferred_element_type=jnp.float32)
        + b_ref[...]
    )


_BM = 2000


def _mm(partials, W, b2d):
    return pl.pallas_call(
        _mm_body,
        grid=(N // _BM,),
        in_specs=[
            pl.BlockSpec((_BM, D), lambda i: (i, 0)),
            pl.BlockSpec((_BM, D), lambda i: (i + N // _BM, 0)),
            pl.BlockSpec((D, D), lambda i: (0, 0)),
            pl.BlockSpec((1, D), lambda i: (0, 0)),
        ],
        out_specs=pl.BlockSpec((_BM, D), lambda i: (i, 0)),
        out_shape=jax.ShapeDtypeStruct((N, D), jnp.float32),
    )(partials, partials, W, b2d)


def kernel(x, adj_t, W, b):
    partials = _sc_aggregate(x, adj_t.reshape(2 * E))
    return _mm(partials, W, b.reshape(1, D))


# speedup vs baseline: 15.4975x; 1.3533x over previous
import functools

import jax
import jax.numpy as jnp
from jax import lax
from jax.experimental import pallas as pl
from jax.experimental.pallas import tpu as pltpu
from jax.experimental.pallas import tpu_sc as plsc

N = 10000
D = 128
E = 320000
NC = 2
NS = 16
NW = NC * NS
E_PER_TILE = E // NW
K = 80
CHUNKS = E_PER_TILE // K
R_MAIN = 624
ZR = 16
NBUF = 3
NIDX = 6


def _sc_body(x_hbm, adj_hbm, part_hbm,
             sidx_v, didx_v, rows_v, zero_v, acc_sh,
             sem_z, si0, si1, si2, si3, si4, si5,
             sg0, sg1, sg2, ss0, ss1, ss2):
    sem_i = (si0, si1, si2, si3, si4, si5)
    sem_g = (sg0, sg1, sg2)
    sem_s = (ss0, ss1, ss2)
    c = lax.axis_index("c")
    s = lax.axis_index("s")

    wid = c * NS + s

    for r in range(ZR):
        for j in range(D // 16):
            zero_v[r, pl.ds(j * 16, 16)] = jnp.zeros((16,), jnp.float32)

    zcopies = []
    for i in range(R_MAIN // ZR):
        zcopies.append(pltpu.async_copy(
            zero_v, acc_sh.at[pl.ds(s * R_MAIN + i * ZR, ZR)], sem_z))

    @pl.when(s == NS - 1)
    def _zero_tail():
        pltpu.async_copy(zero_v, acc_sh.at[pl.ds(N - ZR, ZR)], sem_z).wait()

    for cp in zcopies:
        cp.wait()
    plsc.subcore_barrier()

    base_e = wid * E_PER_TILE

    def idx_fire(chunk, q):
        off = base_e + chunk * K
        pltpu.async_copy(adj_hbm.at[pl.ds(off, K)], sidx_v.at[q], sem_i[q])
        pltpu.async_copy(adj_hbm.at[pl.ds(E + off, K)], didx_v.at[q],
                         sem_i[q])

    def idx_wait(q):
        pltpu.make_async_copy(adj_hbm.at[pl.ds(0, K)], sidx_v.at[q],
                              sem_i[q]).wait()
        pltpu.make_async_copy(adj_hbm.at[pl.ds(0, K)], didx_v.at[q],
                              sem_i[q]).wait()

    def gather_fire(q, b):
        pltpu.async_copy(x_hbm.at[sidx_v.at[q]], rows_v.at[b], sem_g[b])

    def gather_wait(b):
        pltpu.make_async_copy(x_hbm.at[sidx_v.at[0]], rows_v.at[b],
                              sem_g[b]).wait()

    def scatter_fire(q, b):
        pltpu.async_copy(rows_v.at[b], acc_sh.at[didx_v.at[q]],
                         sem_s[b], add=True)

    def scatter_wait(b):
        pltpu.make_async_copy(rows_v.at[b], acc_sh.at[didx_v.at[0]],
                              sem_s[b]).wait()

    def step(cexpr, u, first=False, fire_idx=True):
        b = u % NBUF
        q = u % NIDX
        if not first:
            scatter_wait(b)
        if fire_idx:
            idx_fire(cexpr + NBUF, (u + NBUF) % NIDX)
        idx_wait(q)
        gather_fire(q, b)
        if cexpr is not None and not (first and u == 0):
            gather_wait((u - 1) % NBUF)
            scatter_fire((u - 1) % NIDX, (u - 1) % NBUF)

    for q in range(NBUF):
        idx_fire(q, q)
    for u in range(NIDX):
        step(u, u, first=(u < NBUF))

    def ring(j, carry):
        c0 = j * NIDX
        for u in range(NIDX):
            step(c0 + u, u)
        return carry
    lax.fori_loop(1, CHUNKS // NIDX, ring, 0)

    for u, cc in ((0, 120), (1, 121)):
        step(cc, u)
    for u, cc in ((2, 122), (3, 123), (4, 124)):
        step(cc, u, fire_idx=False)
    gather_wait(4 % NBUF)
    scatter_fire(4 % NIDX, 4 % NBUF)
    for b in (2, 0, 1):
        scatter_wait(b)

    plsc.subcore_barrier()
    pltpu.sync_copy(
        acc_sh.at[pl.ds(s * R_MAIN, R_MAIN)],
        part_hbm.at[pl.ds(c * N + s * R_MAIN, R_MAIN)])

    @pl.when(s == NS - 1)
    def _write_tail():
        pltpu.sync_copy(acc_sh.at[pl.ds(N - ZR, ZR)],
                        part_hbm.at[pl.ds(c * N + N - ZR, ZR)])


_sc_aggregate = functools.partial(
    pl.kernel,
    mesh=plsc.VectorSubcoreMesh(core_axis_name="c", subcore_axis_name="s"),
    out_type=jax.ShapeDtypeStruct((NC * N, D), jnp.float32),
    scratch_types=[
        pltpu.VMEM((NIDX, K), jnp.int32),
        pltpu.VMEM((NIDX, K), jnp.int32),
        pltpu.VMEM((NBUF, K, D), jnp.float32),
        pltpu.VMEM((ZR, D), jnp.float32),
        pltpu.VMEM_SHARED((N, D), jnp.float32),
    ] + [pltpu.SemaphoreType.DMA] * 13,
)(_sc_body)


def _mm_body(p0_ref, p1_ref, w_ref, b_ref, o_ref):
    acc = p0_ref[...] + p1_ref[...]
    o_ref[...] = (
        jnp.dot(acc, w_ref[...], preferred_element_type=jnp.float32)
        + b_ref[...]
    )


_BM = 2000


def _mm(partials, W, b2d):
    return pl.pallas_call(
        _mm_body,
        grid=(N // _BM,),
        in_specs=[
            pl.BlockSpec((_BM, D), lambda i: (i, 0)),
            pl.BlockSpec((_BM, D), lambda i: (i + N // _BM, 0)),
            pl.BlockSpec((D, D), lambda i: (0, 0)),
            pl.BlockSpec((1, D), lambda i: (0, 0)),
        ],
        out_specs=pl.BlockSpec((_BM, D), lambda i: (i, 0)),
        out_shape=jax.ShapeDtypeStruct((N, D), jnp.float32),
    )(partials, partials, W, b2d)


def kernel(x, adj_t, W, b):
    partials = _sc_aggregate(x, adj_t.reshape(2 * E))
    return _mm(partials, W, b.reshape(1, D))
